# baseline (speedup 1.0000x reference)
import jax
import jax.numpy as jnp
from jax.experimental import pallas as pl
from jax.experimental.pallas import tpu as pltpu

_B, _V = 16384, 130
_W = _V * 3
_F = _V - 2
_COUNT = 3 * _F * _B
_BB = 512


def _body(xo_ref, xg_ref, o_ref):
    pid = pl.program_id(0)

    @pl.when(pid == 0)
    def _():
        o_ref[0, 0] = 0.0

    def edge_dists(x):
        e = x[:, 3:_W] - x[:, 0:_W - 3]
        e2 = e * e
        n2 = e2[:, 0:_W - 5] + e2[:, 1:_W - 4] + e2[:, 2:_W - 3]
        f = x[:, 6:_W] - x[:, 0:_W - 6]
        f2 = f * f
        s2 = f2[:, 0:_W - 8] + f2[:, 1:_W - 7] + f2[:, 2:_W - 6]
        return jnp.sqrt(n2), jnp.sqrt(s2)

    no, so = edge_dists(xo_ref[...])
    ng, sg = edge_dists(xg_ref[...])
    dn = jnp.abs(no - ng)
    ds = jnp.abs(so - sg)

    ln = jax.lax.broadcasted_iota(jnp.int32, dn.shape, 1)
    ls = jax.lax.broadcasted_iota(jnp.int32, ds.shape, 1)
    wn = jnp.where(
        ln % 3 == 0,
        jnp.where((ln == 0) | (ln == _W - 6), 1.0, 2.0),
        0.0,
    )
    ws = jnp.where(ls % 3 == 0, 1.0, 0.0)
    part = (jnp.sum(dn * wn) + jnp.sum(ds * ws)) * (1.0 / _COUNT)
    o_ref[0, 0] += part


@jax.jit
def kernel(coord_out, coord_gt):
    xo = coord_out.reshape(_B, _W)
    xg = coord_gt.reshape(_B, _W)
    acc = pl.pallas_call(
        _body,
        grid=(_B // _BB,),
        in_specs=[
            pl.BlockSpec((_BB, _W), lambda i: (i, 0)),
            pl.BlockSpec((_BB, _W), lambda i: (i, 0)),
        ],
        out_specs=pl.BlockSpec(memory_space=pltpu.SMEM),
        out_shape=jax.ShapeDtypeStruct((1, 1), jnp.float32),
        compiler_params=pltpu.CompilerParams(
            dimension_semantics=("arbitrary",)),
    )(xo, xg)
    return acc[0, 0]
